# Initial kernel scaffold; baseline (speedup 1.0000x reference)
import jax
import jax.numpy as jnp
from jax.experimental import pallas as pl


def kernel(query, reference_points, input_flatten, input_spatial_shapes, input_level_start_index, Wv, bv, Wo, bo, Wa, ba, Wout, bout):
    raise NotImplementedError("write your pallas kernel here")


# speedup vs baseline: 74.6481x; 74.6481x over previous
import functools

import numpy as np
import jax
import jax.numpy as jnp
from jax import lax
from jax.experimental import pallas as pl
from jax.experimental.pallas import tpu as pltpu
from jax.experimental.pallas import tpu_sc as plsc

_D = 256
_NH = 8
_NL = 4
_NP = 4
_HD = _D // _NH
_SPAT = np.array([[64, 64], [32, 32], [16, 16], [8, 8]], dtype=np.int64)
_LSTART = np.array([0, 4096, 5120, 5376], dtype=np.int64)
_N = 4
_LQ = 5440
_S = int((_SPAT[:, 0] * _SPAT[:, 1]).sum())
_NQ = _N * _LQ
_NSAMP = _NH * _NL * _NP
_BLK = 128
_GRID = _NQ // _BLK

_NW = 32
_QPW = _NQ // _NW
_CQ = 2
_NCHUNK = _QPW // _CQ

_s = np.arange(_NSAMP)
_lvl = (_s // _NP) % _NL
_head = _s // (_NL * _NP)
_Wl = _SPAT[_lvl, 1].astype(np.int32)
_Hl = _SPAT[_lvl, 0].astype(np.int32)

_WI = _Wl[None, :]
_HI = _Hl[None, :]
_W8 = (_Wl * _NH)[None, :].astype(np.int32)
_CBASE = ((_LSTART[_lvl] * _NH) + _head)[None, :].astype(np.int32)

_RX8 = np.zeros((2 * _NL, _NSAMP), np.float32)
_RY8 = np.zeros((2 * _NL, _NSAMP), np.float32)
for _j in range(_NSAMP):
    _RX8[2 * _lvl[_j], _j] = float(_Wl[_j])
    _RY8[2 * _lvl[_j] + 1, _j] = float(_Hl[_j])

_BD = (( _s[:, None] // (_NL * _NP)) == (_s[None, :] // (_NL * _NP))).astype(np.float32)

_BBASE = ((np.arange(_NQ) // _LQ) * (_S * _NH)).astype(np.int32)[:, None]


def _mm_body(x_ref, w_ref, b_ref, o_ref):
    o_ref[...] = jnp.dot(x_ref[...], w_ref[...],
                         preferred_element_type=jnp.float32, precision=jax.lax.Precision.HIGHEST) + b_ref[...]


def _matmul_bias(x, w, b):
    n, k = x.shape
    m = w.shape[1]
    return pl.pallas_call(
        _mm_body,
        grid=(n // _BLK,),
        in_specs=[
            pl.BlockSpec((_BLK, k), lambda i: (i, 0)),
            pl.BlockSpec((k, m), lambda i: (0, 0)),
            pl.BlockSpec((1, m), lambda i: (0, 0)),
        ],
        out_specs=pl.BlockSpec((_BLK, m), lambda i: (i, 0)),
        out_shape=jax.ShapeDtypeStruct((n, m), jnp.float32),
    )(x, w, b[None, :])


def _prep_body(q_ref, rp_ref, bb_ref, wox_ref, woy_ref, wa_ref,
               box_ref, boy_ref, ba_ref, rx_ref, ry_ref, bd_ref,
               wi_ref, hi_ref, w8_ref, cb_ref,
               idx_ref, wgt_ref):
    q = q_ref[...]
    offx = jnp.dot(q, wox_ref[...], preferred_element_type=jnp.float32, precision=jax.lax.Precision.HIGHEST) + box_ref[...]
    offy = jnp.dot(q, woy_ref[...], preferred_element_type=jnp.float32, precision=jax.lax.Precision.HIGHEST) + boy_ref[...]
    rp = rp_ref[...]
    x = jnp.dot(rp, rx_ref[...], preferred_element_type=jnp.float32, precision=jax.lax.Precision.HIGHEST) + offx - 0.5
    y = jnp.dot(rp, ry_ref[...], preferred_element_type=jnp.float32, precision=jax.lax.Precision.HIGHEST) + offy - 0.5

    logits = jnp.dot(q, wa_ref[...], preferred_element_type=jnp.float32, precision=jax.lax.Precision.HIGHEST) + ba_ref[...]
    m = jnp.max(logits, axis=1, keepdims=True)
    e = jnp.exp(logits - m)
    aw = e / jnp.dot(e, bd_ref[...], preferred_element_type=jnp.float32, precision=jax.lax.Precision.HIGHEST)

    x0 = jnp.floor(x)
    y0 = jnp.floor(y)
    fx = x - x0
    fy = y - y0
    x0i = x0.astype(jnp.int32)
    y0i = y0.astype(jnp.int32)
    wi = wi_ref[...]
    hi = hi_ref[...]
    w8 = w8_ref[...]
    base = bb_ref[...] + cb_ref[...]
    for ci, (dx, dy) in enumerate(((0, 0), (1, 0), (0, 1), (1, 1))):
        xi = x0i + dx
        yi = y0i + dy
        valid = (xi >= 0) & (xi < wi) & (yi >= 0) & (yi < hi)
        xc = jnp.clip(xi, 0, wi - 1)
        yc = jnp.clip(yi, 0, hi - 1)
        idx_ref[:, ci * 128:(ci + 1) * 128] = base + yc * w8 + xc * _NH
        wx = fx if dx else 1.0 - fx
        wy = fy if dy else 1.0 - fy
        wgt_ref[:, ci * 128:(ci + 1) * 128] = jnp.where(valid, wx * wy * aw, 0.0)


def _prep(q2, rp8, consts):
    (bb, wox, woy, wa, box, boy, ba, rx, ry, bd, wi, hi, w8, cb) = consts
    full = lambda a, b: pl.BlockSpec((a, b), lambda i: (0, 0))
    return pl.pallas_call(
        _prep_body,
        grid=(_GRID,),
        in_specs=[
            pl.BlockSpec((_BLK, _D), lambda i: (i, 0)),
            pl.BlockSpec((_BLK, 8), lambda i: (i, 0)),
            pl.BlockSpec((_BLK, 1), lambda i: (i, 0)),
            full(_D, 128), full(_D, 128), full(_D, 128),
            full(1, 128), full(1, 128), full(1, 128),
            full(8, 128), full(8, 128), full(128, 128),
            full(1, 128), full(1, 128), full(1, 128), full(1, 128),
        ],
        out_specs=[
            pl.BlockSpec((_BLK, 512), lambda i: (i, 0)),
            pl.BlockSpec((_BLK, 512), lambda i: (i, 0)),
        ],
        out_shape=[
            jax.ShapeDtypeStruct((_NQ, 512), jnp.int32),
            jax.ShapeDtypeStruct((_NQ, 512), jnp.float32),
        ],
    )(q2, rp8, bb, wox, woy, wa, box, boy, ba, rx, ry, bd, wi, hi, w8, cb)


def _sc_body(table_hbm, idx_hbm, wgt_hbm, out_hbm, idx_v, wgt_v, rows_v, out_v, sem):
    wid = lax.axis_index("s") * 2 + lax.axis_index("c")
    q_base = wid * _QPW

    def chunk(i, carry):
        q0 = q_base + i * _CQ
        pltpu.sync_copy(idx_hbm.at[pl.ds(q0 * 4, _CQ * 4)], idx_v)
        pltpu.sync_copy(wgt_hbm.at[pl.ds(q0, _CQ)], wgt_v)
        cps = [
            pltpu.async_copy(table_hbm.at[idx_v.at[g]],
                             rows_v.at[pl.ds(g * 128, 128)], sem)
            for g in range(_CQ * 4)
        ]
        for c in cps:
            c.wait()
        for q in range(_CQ):
            def hbody(h, c2):
                acc0 = jnp.zeros((16,), jnp.float32)
                acc1 = jnp.zeros((16,), jnp.float32)
                for c in range(4):
                    wv = wgt_v[q, pl.ds(pl.multiple_of(c * 128 + h * 16, 16), 16)]
                    for k2 in range(16):
                        r = q * 512 + c * 128 + h * 16 + k2
                        w = wv[k2]
                        acc0 = acc0 + w * rows_v[r, pl.ds(0, 16)]
                        acc1 = acc1 + w * rows_v[r, pl.ds(16, 16)]
                o = pl.multiple_of(h * 32, 32)
                out_v[q, pl.ds(o, 16)] = acc0
                out_v[q, pl.ds(o + 16, 16)] = acc1
                return c2
            lax.fori_loop(0, _NH, hbody, 0)
        pltpu.sync_copy(out_v, out_hbm.at[pl.ds(q0, _CQ)])
        return carry

    lax.fori_loop(0, _NCHUNK, chunk, 0)


def _sc_sample(table, idx4, wgt):
    mesh = plsc.VectorSubcoreMesh(core_axis_name="c", subcore_axis_name="s")
    return pl.kernel(
        _sc_body,
        out_type=jax.ShapeDtypeStruct((_NQ, _D), jnp.float32),
        mesh=mesh,
        compiler_params=pltpu.CompilerParams(use_tc_tiling_on_sc=False),
        scratch_types=[
            pltpu.VMEM((_CQ * 4, 128), jnp.int32),
            pltpu.VMEM((_CQ, 512), jnp.float32),
            pltpu.VMEM((_CQ * 512, _HD), jnp.float32),
            pltpu.VMEM((_CQ, _D), jnp.float32),
            pltpu.SemaphoreType.DMA,
        ],
    )(table, idx4, wgt)


def kernel(query, reference_points, input_flatten, input_spatial_shapes,
           input_level_start_index, Wv, bv, Wo, bo, Wa, ba, Wout, bout):
    q2 = query.reshape(_NQ, _D)
    rp8 = reference_points.reshape(_NQ, _NL * 2)

    value = _matmul_bias(input_flatten.reshape(_N * _S, _D), Wv, bv)
    table = value.reshape(_N * _S * _NH, _HD)

    consts = (
        jnp.asarray(_BBASE),
        Wo[:, 0::2], Wo[:, 1::2], Wa,
        bo[0::2][None, :], bo[1::2][None, :], ba[None, :],
        jnp.asarray(_RX8), jnp.asarray(_RY8), jnp.asarray(_BD),
        jnp.asarray(_WI), jnp.asarray(_HI), jnp.asarray(_W8),
        jnp.asarray(_CBASE),
    )
    idx, wgt = _prep(q2, rp8, consts)

    sampled = _sc_sample(table, idx.reshape(_NQ * 4, 128), wgt)
    out = _matmul_bias(sampled, Wout, bout)
    return out.reshape(_N, _LQ, _D)


# speedup vs baseline: 115.3262x; 1.5449x over previous
import functools

import numpy as np
import jax
import jax.numpy as jnp
from jax import lax
from jax.experimental import pallas as pl
from jax.experimental.pallas import tpu as pltpu
from jax.experimental.pallas import tpu_sc as plsc

_D = 256
_NH = 8
_NL = 4
_NP = 4
_HD = _D // _NH
_SPAT = np.array([[64, 64], [32, 32], [16, 16], [8, 8]], dtype=np.int64)
_LSTART = np.array([0, 4096, 5120, 5376], dtype=np.int64)
_N = 4
_LQ = 5440
_S = int((_SPAT[:, 0] * _SPAT[:, 1]).sum())
_NQ = _N * _LQ
_NSAMP = _NH * _NL * _NP
_BLK = 128
_GRID = _NQ // _BLK

_NW = 32
_QPW = _NQ // _NW
_CQ = 2
_NCHUNK = _QPW // _CQ

_s = np.arange(_NSAMP)
_lvl = (_s // _NP) % _NL
_head = _s // (_NL * _NP)
_Wl = _SPAT[_lvl, 1].astype(np.int32)
_Hl = _SPAT[_lvl, 0].astype(np.int32)

_WI = _Wl[None, :]
_HI = _Hl[None, :]
_W8 = (_Wl * _NH)[None, :].astype(np.int32)
_CBASE = ((_LSTART[_lvl] * _NH) + _head)[None, :].astype(np.int32)

_RX8 = np.zeros((2 * _NL, _NSAMP), np.float32)
_RY8 = np.zeros((2 * _NL, _NSAMP), np.float32)
for _j in range(_NSAMP):
    _RX8[2 * _lvl[_j], _j] = float(_Wl[_j])
    _RY8[2 * _lvl[_j] + 1, _j] = float(_Hl[_j])

_BD = (( _s[:, None] // (_NL * _NP)) == (_s[None, :] // (_NL * _NP))).astype(np.float32)

_BBASE = ((np.arange(_NQ) // _LQ) * (_S * _NH)).astype(np.int32)[:, None]


def _mm_body(x_ref, w_ref, b_ref, o_ref):
    o_ref[...] = jnp.dot(x_ref[...], w_ref[...],
                         preferred_element_type=jnp.float32, precision=jax.lax.Precision.HIGHEST) + b_ref[...]


def _matmul_bias(x, w, b):
    n, k = x.shape
    m = w.shape[1]
    return pl.pallas_call(
        _mm_body,
        grid=(n // _BLK,),
        in_specs=[
            pl.BlockSpec((_BLK, k), lambda i: (i, 0)),
            pl.BlockSpec((k, m), lambda i: (0, 0)),
            pl.BlockSpec((1, m), lambda i: (0, 0)),
        ],
        out_specs=pl.BlockSpec((_BLK, m), lambda i: (i, 0)),
        out_shape=jax.ShapeDtypeStruct((n, m), jnp.float32),
    )(x, w, b[None, :])


def _prep_body(q_ref, rp_ref, bb_ref, wox_ref, woy_ref, wa_ref,
               box_ref, boy_ref, ba_ref, rx_ref, ry_ref, bd_ref,
               wi_ref, hi_ref, w8_ref, cb_ref,
               idx_ref, wgt_ref):
    q = q_ref[...]
    offx = jnp.dot(q, wox_ref[...], preferred_element_type=jnp.float32, precision=jax.lax.Precision.HIGHEST) + box_ref[...]
    offy = jnp.dot(q, woy_ref[...], preferred_element_type=jnp.float32, precision=jax.lax.Precision.HIGHEST) + boy_ref[...]
    rp = rp_ref[...]
    x = jnp.dot(rp, rx_ref[...], preferred_element_type=jnp.float32, precision=jax.lax.Precision.HIGHEST) + offx - 0.5
    y = jnp.dot(rp, ry_ref[...], preferred_element_type=jnp.float32, precision=jax.lax.Precision.HIGHEST) + offy - 0.5

    logits = jnp.dot(q, wa_ref[...], preferred_element_type=jnp.float32, precision=jax.lax.Precision.HIGHEST) + ba_ref[...]
    m = jnp.max(logits, axis=1, keepdims=True)
    e = jnp.exp(logits - m)
    aw = e / jnp.dot(e, bd_ref[...], preferred_element_type=jnp.float32, precision=jax.lax.Precision.HIGHEST)

    x0 = jnp.floor(x)
    y0 = jnp.floor(y)
    fx = x - x0
    fy = y - y0
    x0i = x0.astype(jnp.int32)
    y0i = y0.astype(jnp.int32)
    wi = wi_ref[...]
    hi = hi_ref[...]
    w8 = w8_ref[...]
    base = bb_ref[...] + cb_ref[...]
    for ci, (dx, dy) in enumerate(((0, 0), (1, 0), (0, 1), (1, 1))):
        xi = x0i + dx
        yi = y0i + dy
        valid = (xi >= 0) & (xi < wi) & (yi >= 0) & (yi < hi)
        xc = jnp.clip(xi, 0, wi - 1)
        yc = jnp.clip(yi, 0, hi - 1)
        idx_ref[:, ci * 128:(ci + 1) * 128] = base + yc * w8 + xc * _NH
        wx = fx if dx else 1.0 - fx
        wy = fy if dy else 1.0 - fy
        wgt_ref[:, ci * 128:(ci + 1) * 128] = jnp.where(valid, wx * wy * aw, 0.0)


def _prep(q2, rp8, consts):
    (bb, wox, woy, wa, box, boy, ba, rx, ry, bd, wi, hi, w8, cb) = consts
    full = lambda a, b: pl.BlockSpec((a, b), lambda i: (0, 0))
    return pl.pallas_call(
        _prep_body,
        grid=(_GRID,),
        in_specs=[
            pl.BlockSpec((_BLK, _D), lambda i: (i, 0)),
            pl.BlockSpec((_BLK, 8), lambda i: (i, 0)),
            pl.BlockSpec((_BLK, 1), lambda i: (i, 0)),
            full(_D, 128), full(_D, 128), full(_D, 128),
            full(1, 128), full(1, 128), full(1, 128),
            full(8, 128), full(8, 128), full(128, 128),
            full(1, 128), full(1, 128), full(1, 128), full(1, 128),
        ],
        out_specs=[
            pl.BlockSpec((_BLK, 512), lambda i: (i, 0)),
            pl.BlockSpec((_BLK, 512), lambda i: (i, 0)),
        ],
        out_shape=[
            jax.ShapeDtypeStruct((_NQ, 512), jnp.int32),
            jax.ShapeDtypeStruct((_NQ, 512), jnp.float32),
        ],
    )(q2, rp8, bb, wox, woy, wa, box, boy, ba, rx, ry, bd, wi, hi, w8, cb)


def _sc_body(table_hbm, idx_hbm, wgt_hbm, out_hbm,
             idx0, idx1, wgt0, wgt1, rows0, rows1, out0, out1,
             sg0, sg1, sio0, sio1, so0, so1):
    idx_v = (idx0, idx1)
    wgt_v = (wgt0, wgt1)
    rows_v = (rows0, rows1)
    out_v = (out0, out1)
    sg = (sg0, sg1)
    sio = (sio0, sio1)
    so = (so0, so1)
    wid = lax.axis_index("s") * 2 + lax.axis_index("c")
    q_base = wid * _QPW
    qmax = _NQ - _CQ

    def qof(c):
        return jnp.minimum(q_base + c * _CQ, qmax)

    def issue_io(c, b):
        q = qof(c)
        pltpu.async_copy(idx_hbm.at[pl.ds(q * 4, _CQ * 4)], idx_v[b], sio[b])
        pltpu.async_copy(wgt_hbm.at[pl.ds(q, _CQ)], wgt_v[b], sio[b])

    def wait_io(b):
        pltpu.make_async_copy(idx_hbm.at[pl.ds(0, _CQ * 4)], idx_v[b], sio[b]).wait()
        pltpu.make_async_copy(wgt_hbm.at[pl.ds(0, _CQ)], wgt_v[b], sio[b]).wait()

    def fire_gathers(b):
        for g in range(_CQ * 4):
            pltpu.async_copy(table_hbm.at[idx_v[b].at[g]],
                             rows_v[b].at[pl.ds(g * 128, 128)], sg[b])

    def wait_gathers(b):
        pltpu.make_async_copy(table_hbm.at[pl.ds(0, _CQ * 512)], rows_v[b], sg[b]).wait()

    def wait_out(b):
        pltpu.make_async_copy(out_v[b], out_hbm.at[pl.ds(q_base, _CQ)], so[b]).wait()

    def compute(b, q0):
        for q in range(_CQ):
            def hbody(h, c2):
                acc0 = jnp.zeros((16,), jnp.float32)
                acc1 = jnp.zeros((16,), jnp.float32)
                for c in range(4):
                    wv = wgt_v[b][q, pl.ds(pl.multiple_of(c * 128 + h * 16, 16), 16)]
                    for k2 in range(16):
                        r = q * 512 + c * 128 + h * 16 + k2
                        w = wv[k2]
                        acc0 = acc0 + w * rows_v[b][r, pl.ds(0, 16)]
                        acc1 = acc1 + w * rows_v[b][r, pl.ds(16, 16)]
                o = pl.multiple_of(h * 32, 32)
                out_v[b][q, pl.ds(o, 16)] = acc0
                out_v[b][q, pl.ds(o + 16, 16)] = acc1
                return c2
            lax.fori_loop(0, _NH, hbody, 0)
        pltpu.async_copy(out_v[b], out_hbm.at[pl.ds(q0, _CQ)], so[b])

    issue_io(0, 0)
    wait_io(0)
    fire_gathers(0)
    issue_io(1, 1)

    def iter_body(i, carry):
        for b in (0, 1):
            c = 2 * i + b
            wait_io(1 - b)
            fire_gathers(1 - b)
            wait_gathers(b)
            @pl.when(i >= 1)
            def _():
                wait_out(b)
            compute(b, q_base + c * _CQ)
            issue_io(c + 2, b)
        return carry

    lax.fori_loop(0, _NCHUNK // 2, iter_body, 0)

    wait_io(1)
    wait_gathers(0)
    wait_out(0)
    wait_out(1)


def _sc_sample(table, idx4, wgt):
    mesh = plsc.VectorSubcoreMesh(core_axis_name="c", subcore_axis_name="s")
    return pl.kernel(
        _sc_body,
        out_type=jax.ShapeDtypeStruct((_NQ, _D), jnp.float32),
        mesh=mesh,
        compiler_params=pltpu.CompilerParams(use_tc_tiling_on_sc=False),
        scratch_types=[
            pltpu.VMEM((_CQ * 4, 128), jnp.int32),
            pltpu.VMEM((_CQ * 4, 128), jnp.int32),
            pltpu.VMEM((_CQ, 512), jnp.float32),
            pltpu.VMEM((_CQ, 512), jnp.float32),
            pltpu.VMEM((_CQ * 512, _HD), jnp.float32),
            pltpu.VMEM((_CQ * 512, _HD), jnp.float32),
            pltpu.VMEM((_CQ, _D), jnp.float32),
            pltpu.VMEM((_CQ, _D), jnp.float32),
            pltpu.SemaphoreType.DMA,
            pltpu.SemaphoreType.DMA,
            pltpu.SemaphoreType.DMA,
            pltpu.SemaphoreType.DMA,
            pltpu.SemaphoreType.DMA,
            pltpu.SemaphoreType.DMA,
        ],
    )(table, idx4, wgt)


def kernel(query, reference_points, input_flatten, input_spatial_shapes,
           input_level_start_index, Wv, bv, Wo, bo, Wa, ba, Wout, bout):
    q2 = query.reshape(_NQ, _D)
    rp8 = reference_points.reshape(_NQ, _NL * 2)

    value = _matmul_bias(input_flatten.reshape(_N * _S, _D), Wv, bv)
    table = value.reshape(_N * _S * _NH, _HD)

    consts = (
        jnp.asarray(_BBASE),
        Wo[:, 0::2], Wo[:, 1::2], Wa,
        bo[0::2][None, :], bo[1::2][None, :], ba[None, :],
        jnp.asarray(_RX8), jnp.asarray(_RY8), jnp.asarray(_BD),
        jnp.asarray(_WI), jnp.asarray(_HI), jnp.asarray(_W8),
        jnp.asarray(_CBASE),
    )
    idx, wgt = _prep(q2, rp8, consts)

    sampled = _sc_sample(table, idx.reshape(_NQ * 4, 128), wgt)
    out = _matmul_bias(sampled, Wout, bout)
    return out.reshape(_N, _LQ, _D)


# speedup vs baseline: 119.0853x; 1.0326x over previous
import functools

import numpy as np
import jax
import jax.numpy as jnp
from jax import lax
from jax.experimental import pallas as pl
from jax.experimental.pallas import tpu as pltpu
from jax.experimental.pallas import tpu_sc as plsc

_D = 256
_NH = 8
_NL = 4
_NP = 4
_HD = _D // _NH
_SPAT = np.array([[64, 64], [32, 32], [16, 16], [8, 8]], dtype=np.int64)
_LSTART = np.array([0, 4096, 5120, 5376], dtype=np.int64)
_N = 4
_LQ = 5440
_S = int((_SPAT[:, 0] * _SPAT[:, 1]).sum())
_NQ = _N * _LQ
_NSAMP = _NH * _NL * _NP
_BLK = 128
_GRID = _NQ // _BLK

_NW = 32
_QPW = _NQ // _NW
_CQ = 2
_NCHUNK = _QPW // _CQ

_s = np.arange(_NSAMP)
_lvl = (_s // _NP) % _NL
_head = _s // (_NL * _NP)
_Wl = _SPAT[_lvl, 1].astype(np.int32)
_Hl = _SPAT[_lvl, 0].astype(np.int32)

_WI = _Wl[None, :]
_HI = _Hl[None, :]
_W8 = (_Wl * _NH)[None, :].astype(np.int32)
_CBASE = ((_LSTART[_lvl] * _NH) + _head)[None, :].astype(np.int32)

_RX8 = np.zeros((2 * _NL, _NSAMP), np.float32)
_RY8 = np.zeros((2 * _NL, _NSAMP), np.float32)
for _j in range(_NSAMP):
    _RX8[2 * _lvl[_j], _j] = float(_Wl[_j])
    _RY8[2 * _lvl[_j] + 1, _j] = float(_Hl[_j])

_BD = (( _s[:, None] // (_NL * _NP)) == (_s[None, :] // (_NL * _NP))).astype(np.float32)

_BBASE = ((np.arange(_NQ) // _LQ) * (_S * _NH)).astype(np.int32)[:, None]


def _mm_body(x_ref, w_ref, b_ref, o_ref):
    o_ref[...] = jnp.dot(x_ref[...], w_ref[...],
                         preferred_element_type=jnp.float32) + b_ref[...]


def _matmul_bias(x, w, b):
    n, k = x.shape
    m = w.shape[1]
    return pl.pallas_call(
        _mm_body,
        grid=(n // _BLK,),
        in_specs=[
            pl.BlockSpec((_BLK, k), lambda i: (i, 0)),
            pl.BlockSpec((k, m), lambda i: (0, 0)),
            pl.BlockSpec((1, m), lambda i: (0, 0)),
        ],
        out_specs=pl.BlockSpec((_BLK, m), lambda i: (i, 0)),
        out_shape=jax.ShapeDtypeStruct((n, m), jnp.float32),
    )(x, w, b[None, :])


def _prep_body(q_ref, rp_ref, bb_ref, wox_ref, woy_ref, wa_ref,
               box_ref, boy_ref, ba_ref, rx_ref, ry_ref, bd_ref,
               wi_ref, hi_ref, w8_ref, cb_ref,
               idx_ref, wgt_ref):
    q = q_ref[...]
    offx = jnp.dot(q, wox_ref[...], preferred_element_type=jnp.float32, precision=jax.lax.Precision.HIGHEST) + box_ref[...]
    offy = jnp.dot(q, woy_ref[...], preferred_element_type=jnp.float32, precision=jax.lax.Precision.HIGHEST) + boy_ref[...]
    rp = rp_ref[...]
    x = jnp.dot(rp, rx_ref[...], preferred_element_type=jnp.float32, precision=jax.lax.Precision.HIGHEST) + offx - 0.5
    y = jnp.dot(rp, ry_ref[...], preferred_element_type=jnp.float32, precision=jax.lax.Precision.HIGHEST) + offy - 0.5

    logits = jnp.dot(q, wa_ref[...], preferred_element_type=jnp.float32) + ba_ref[...]
    m = jnp.max(logits, axis=1, keepdims=True)
    e = jnp.exp(logits - m)
    aw = e / jnp.dot(e, bd_ref[...], preferred_element_type=jnp.float32)

    x0 = jnp.floor(x)
    y0 = jnp.floor(y)
    fx = x - x0
    fy = y - y0
    x0i = x0.astype(jnp.int32)
    y0i = y0.astype(jnp.int32)
    wi = wi_ref[...]
    hi = hi_ref[...]
    w8 = w8_ref[...]
    base = bb_ref[...] + cb_ref[...]
    for ci, (dx, dy) in enumerate(((0, 0), (1, 0), (0, 1), (1, 1))):
        xi = x0i + dx
        yi = y0i + dy
        valid = (xi >= 0) & (xi < wi) & (yi >= 0) & (yi < hi)
        xc = jnp.clip(xi, 0, wi - 1)
        yc = jnp.clip(yi, 0, hi - 1)
        idx_ref[:, ci * 128:(ci + 1) * 128] = base + yc * w8 + xc * _NH
        wx = fx if dx else 1.0 - fx
        wy = fy if dy else 1.0 - fy
        wgt_ref[:, ci * 128:(ci + 1) * 128] = jnp.where(valid, wx * wy * aw, 0.0)


def _prep(q2, rp8, consts):
    (bb, wox, woy, wa, box, boy, ba, rx, ry, bd, wi, hi, w8, cb) = consts
    full = lambda a, b: pl.BlockSpec((a, b), lambda i: (0, 0))
    return pl.pallas_call(
        _prep_body,
        grid=(_GRID,),
        in_specs=[
            pl.BlockSpec((_BLK, _D), lambda i: (i, 0)),
            pl.BlockSpec((_BLK, 8), lambda i: (i, 0)),
            pl.BlockSpec((_BLK, 1), lambda i: (i, 0)),
            full(_D, 128), full(_D, 128), full(_D, 128),
            full(1, 128), full(1, 128), full(1, 128),
            full(8, 128), full(8, 128), full(128, 128),
            full(1, 128), full(1, 128), full(1, 128), full(1, 128),
        ],
        out_specs=[
            pl.BlockSpec((_BLK, 512), lambda i: (i, 0)),
            pl.BlockSpec((_BLK, 512), lambda i: (i, 0)),
        ],
        out_shape=[
            jax.ShapeDtypeStruct((_NQ, 512), jnp.int32),
            jax.ShapeDtypeStruct((_NQ, 512), jnp.float32),
        ],
    )(q2, rp8, bb, wox, woy, wa, box, boy, ba, rx, ry, bd, wi, hi, w8, cb)


def _sc_body(table_hbm, idx_hbm, wgt_hbm, out_hbm,
             idx0, idx1, wgt0, wgt1, rows0, rows1, out0, out1,
             sg0, sg1, sio0, sio1, so0, so1):
    idx_v = (idx0, idx1)
    wgt_v = (wgt0, wgt1)
    rows_v = (rows0, rows1)
    out_v = (out0, out1)
    sg = (sg0, sg1)
    sio = (sio0, sio1)
    so = (so0, so1)
    wid = lax.axis_index("s") * 2 + lax.axis_index("c")
    q_base = wid * _QPW
    qmax = _NQ - _CQ

    def qof(c):
        return jnp.minimum(q_base + c * _CQ, qmax)

    def issue_io(c, b):
        q = qof(c)
        pltpu.async_copy(idx_hbm.at[pl.ds(q * 4, _CQ * 4)], idx_v[b], sio[b])
        pltpu.async_copy(wgt_hbm.at[pl.ds(q, _CQ)], wgt_v[b], sio[b])

    def wait_io(b):
        pltpu.make_async_copy(idx_hbm.at[pl.ds(0, _CQ * 4)], idx_v[b], sio[b]).wait()
        pltpu.make_async_copy(wgt_hbm.at[pl.ds(0, _CQ)], wgt_v[b], sio[b]).wait()

    def fire_gathers(b):
        for g in range(_CQ * 4):
            pltpu.async_copy(table_hbm.at[idx_v[b].at[g]],
                             rows_v[b].at[pl.ds(g * 128, 128)], sg[b])

    def wait_gathers(b):
        pltpu.make_async_copy(table_hbm.at[pl.ds(0, _CQ * 512)], rows_v[b], sg[b]).wait()

    def wait_out(b):
        pltpu.make_async_copy(out_v[b], out_hbm.at[pl.ds(q_base, _CQ)], so[b]).wait()

    def compute(b, q0):
        for q in range(_CQ):
            def hbody(h, c2):
                acc0 = jnp.zeros((16,), jnp.float32)
                acc1 = jnp.zeros((16,), jnp.float32)
                for c in range(4):
                    wv = wgt_v[b][q, pl.ds(pl.multiple_of(c * 128 + h * 16, 16), 16)]
                    for k2 in range(16):
                        r = q * 512 + c * 128 + h * 16 + k2
                        w = wv[k2]
                        acc0 = acc0 + w * rows_v[b][r, pl.ds(0, 16)]
                        acc1 = acc1 + w * rows_v[b][r, pl.ds(16, 16)]
                o = pl.multiple_of(h * 32, 32)
                out_v[b][q, pl.ds(o, 16)] = acc0
                out_v[b][q, pl.ds(o + 16, 16)] = acc1
                return c2
            lax.fori_loop(0, _NH, hbody, 0)
        pltpu.async_copy(out_v[b], out_hbm.at[pl.ds(q0, _CQ)], so[b])

    issue_io(0, 0)
    wait_io(0)
    fire_gathers(0)
    issue_io(1, 1)

    def iter_body(i, carry):
        for b in (0, 1):
            c = 2 * i + b
            wait_io(1 - b)
            fire_gathers(1 - b)
            wait_gathers(b)
            @pl.when(i >= 1)
            def _():
                wait_out(b)
            compute(b, q_base + c * _CQ)
            issue_io(c + 2, b)
        return carry

    lax.fori_loop(0, _NCHUNK // 2, iter_body, 0)

    wait_io(1)
    wait_gathers(0)
    wait_out(0)
    wait_out(1)


def _sc_sample(table, idx4, wgt):
    mesh = plsc.VectorSubcoreMesh(core_axis_name="c", subcore_axis_name="s")
    return pl.kernel(
        _sc_body,
        out_type=jax.ShapeDtypeStruct((_NQ, _D), jnp.float32),
        mesh=mesh,
        compiler_params=pltpu.CompilerParams(use_tc_tiling_on_sc=False),
        scratch_types=[
            pltpu.VMEM((_CQ * 4, 128), jnp.int32),
            pltpu.VMEM((_CQ * 4, 128), jnp.int32),
            pltpu.VMEM((_CQ, 512), jnp.float32),
            pltpu.VMEM((_CQ, 512), jnp.float32),
            pltpu.VMEM((_CQ * 512, _HD), jnp.float32),
            pltpu.VMEM((_CQ * 512, _HD), jnp.float32),
            pltpu.VMEM((_CQ, _D), jnp.float32),
            pltpu.VMEM((_CQ, _D), jnp.float32),
            pltpu.SemaphoreType.DMA,
            pltpu.SemaphoreType.DMA,
            pltpu.SemaphoreType.DMA,
            pltpu.SemaphoreType.DMA,
            pltpu.SemaphoreType.DMA,
            pltpu.SemaphoreType.DMA,
        ],
    )(table, idx4, wgt)


def kernel(query, reference_points, input_flatten, input_spatial_shapes,
           input_level_start_index, Wv, bv, Wo, bo, Wa, ba, Wout, bout):
    q2 = query.reshape(_NQ, _D)
    rp8 = reference_points.reshape(_NQ, _NL * 2)

    value = _matmul_bias(input_flatten.reshape(_N * _S, _D), Wv, bv)
    table = value.reshape(_N * _S * _NH, _HD)

    consts = (
        jnp.asarray(_BBASE),
        Wo[:, 0::2], Wo[:, 1::2], Wa,
        bo[0::2][None, :], bo[1::2][None, :], ba[None, :],
        jnp.asarray(_RX8), jnp.asarray(_RY8), jnp.asarray(_BD),
        jnp.asarray(_WI), jnp.asarray(_HI), jnp.asarray(_W8),
        jnp.asarray(_CBASE),
    )
    idx, wgt = _prep(q2, rp8, consts)

    sampled = _sc_sample(table, idx.reshape(_NQ * 4, 128), wgt)
    out = _matmul_bias(sampled, Wout, bout)
    return out.reshape(_N, _LQ, _D)


# speedup vs baseline: 125.9830x; 1.0579x over previous
import functools

import numpy as np
import jax
import jax.numpy as jnp
from jax import lax
from jax.experimental import pallas as pl
from jax.experimental.pallas import tpu as pltpu
from jax.experimental.pallas import tpu_sc as plsc

_D = 256
_NH = 8
_NL = 4
_NP = 4
_HD = _D // _NH
_SPAT = np.array([[64, 64], [32, 32], [16, 16], [8, 8]], dtype=np.int64)
_LSTART = np.array([0, 4096, 5120, 5376], dtype=np.int64)
_N = 4
_LQ = 5440
_S = int((_SPAT[:, 0] * _SPAT[:, 1]).sum())
_NQ = _N * _LQ
_NSAMP = _NH * _NL * _NP
_BLK = 128
_GRID = _NQ // _BLK

_NW = 32
_QPW = _NQ // _NW
_CQ = 2
_NCHUNK = _QPW // _CQ

_s = np.arange(_NSAMP)
_lvl = (_s // _NP) % _NL
_head = _s // (_NL * _NP)
_Wl = _SPAT[_lvl, 1].astype(np.int32)
_Hl = _SPAT[_lvl, 0].astype(np.int32)

_WI = _Wl[None, :]
_HI = _Hl[None, :]
_W8 = (_Wl * _NH)[None, :].astype(np.int32)
_CBASE = ((_LSTART[_lvl] * _NH) + _head)[None, :].astype(np.int32)

_RX8 = np.zeros((2 * _NL, _NSAMP), np.float32)
_RY8 = np.zeros((2 * _NL, _NSAMP), np.float32)
for _j in range(_NSAMP):
    _RX8[2 * _lvl[_j], _j] = float(_Wl[_j])
    _RY8[2 * _lvl[_j] + 1, _j] = float(_Hl[_j])

_BD = (( _s[:, None] // (_NL * _NP)) == (_s[None, :] // (_NL * _NP))).astype(np.float32)

_BBASE = ((np.arange(_NQ) // _LQ) * (_S * _NH)).astype(np.int32)[:, None]

_PERM = np.empty(_D, np.int32)
for _k in range(_D):
    _h, _j = divmod(_k, _HD)
    _PERM[_k] = _h * _HD + (2 * _j if _j < 16 else 2 * (_j - 16) + 1)


def _mm_body(x_ref, w_ref, b_ref, o_ref):
    o_ref[...] = (jnp.dot(x_ref[...], w_ref[...],
                          preferred_element_type=jnp.float32)
                  + b_ref[...]).astype(o_ref.dtype)


def _matmul_bias(x, w, b, out_dtype=jnp.float32):
    n, k = x.shape
    m = w.shape[1]
    return pl.pallas_call(
        _mm_body,
        grid=(n // _BLK,),
        in_specs=[
            pl.BlockSpec((_BLK, k), lambda i: (i, 0)),
            pl.BlockSpec((k, m), lambda i: (0, 0)),
            pl.BlockSpec((1, m), lambda i: (0, 0)),
        ],
        out_specs=pl.BlockSpec((_BLK, m), lambda i: (i, 0)),
        out_shape=jax.ShapeDtypeStruct((n, m), out_dtype),
    )(x, w, b[None, :])


def _prep_body(q_ref, rp_ref, bb_ref, wox_ref, woy_ref, wa_ref,
               box_ref, boy_ref, ba_ref, rx_ref, ry_ref, bd_ref,
               wi_ref, hi_ref, w8_ref, cb_ref,
               idx_ref, wgt_ref):
    q = q_ref[...]
    offx = jnp.dot(q, wox_ref[...], preferred_element_type=jnp.float32, precision=jax.lax.Precision.HIGHEST) + box_ref[...]
    offy = jnp.dot(q, woy_ref[...], preferred_element_type=jnp.float32, precision=jax.lax.Precision.HIGHEST) + boy_ref[...]
    rp = rp_ref[...]
    x = jnp.dot(rp, rx_ref[...], preferred_element_type=jnp.float32, precision=jax.lax.Precision.HIGHEST) + offx - 0.5
    y = jnp.dot(rp, ry_ref[...], preferred_element_type=jnp.float32, precision=jax.lax.Precision.HIGHEST) + offy - 0.5

    logits = jnp.dot(q, wa_ref[...], preferred_element_type=jnp.float32) + ba_ref[...]
    m = jnp.max(logits, axis=1, keepdims=True)
    e = jnp.exp(logits - m)
    aw = e / jnp.dot(e, bd_ref[...], preferred_element_type=jnp.float32)

    x0 = jnp.floor(x)
    y0 = jnp.floor(y)
    fx = x - x0
    fy = y - y0
    x0i = x0.astype(jnp.int32)
    y0i = y0.astype(jnp.int32)
    wi = wi_ref[...]
    hi = hi_ref[...]
    w8 = w8_ref[...]
    base = bb_ref[...] + cb_ref[...]
    for ci, (dx, dy) in enumerate(((0, 0), (1, 0), (0, 1), (1, 1))):
        xi = x0i + dx
        yi = y0i + dy
        valid = (xi >= 0) & (xi < wi) & (yi >= 0) & (yi < hi)
        xc = jnp.clip(xi, 0, wi - 1)
        yc = jnp.clip(yi, 0, hi - 1)
        idx_ref[:, ci * 128:(ci + 1) * 128] = base + yc * w8 + xc * _NH
        wx = fx if dx else 1.0 - fx
        wy = fy if dy else 1.0 - fy
        wgt_ref[:, ci * 128:(ci + 1) * 128] = jnp.where(valid, wx * wy * aw, 0.0)


def _prep(q2, rp8, consts):
    (bb, wox, woy, wa, box, boy, ba, rx, ry, bd, wi, hi, w8, cb) = consts
    full = lambda a, b: pl.BlockSpec((a, b), lambda i: (0, 0))
    return pl.pallas_call(
        _prep_body,
        grid=(_GRID,),
        in_specs=[
            pl.BlockSpec((_BLK, _D), lambda i: (i, 0)),
            pl.BlockSpec((_BLK, 8), lambda i: (i, 0)),
            pl.BlockSpec((_BLK, 1), lambda i: (i, 0)),
            full(_D, 128), full(_D, 128), full(_D, 128),
            full(1, 128), full(1, 128), full(1, 128),
            full(8, 128), full(8, 128), full(128, 128),
            full(1, 128), full(1, 128), full(1, 128), full(1, 128),
        ],
        out_specs=[
            pl.BlockSpec((_BLK, 512), lambda i: (i, 0)),
            pl.BlockSpec((_BLK, 512), lambda i: (i, 0)),
        ],
        out_shape=[
            jax.ShapeDtypeStruct((_NQ, 512), jnp.int32),
            jax.ShapeDtypeStruct((_NQ, 512), jnp.float32),
        ],
    )(q2, rp8, bb, wox, woy, wa, box, boy, ba, rx, ry, bd, wi, hi, w8, cb)


def _sc_body(table_hbm, idx_hbm, wgt_hbm, out_hbm,
             idx0, idx1, wgt0, wgt1, rows0, rows1, out0, out1,
             sg0, sg1, sio0, sio1, so0, so1):
    idx_v = (idx0, idx1)
    wgt_v = (wgt0, wgt1)
    rows_v = (rows0, rows1)
    out_v = (out0, out1)
    sg = (sg0, sg1)
    sio = (sio0, sio1)
    so = (so0, so1)
    wid = lax.axis_index("s") * 2 + lax.axis_index("c")
    q_base = wid * _QPW
    qmax = _NQ - _CQ

    def qof(c):
        return jnp.minimum(q_base + c * _CQ, qmax)

    def issue_io(c, b):
        q = qof(c)
        pltpu.async_copy(idx_hbm.at[pl.ds(q * 4, _CQ * 4)], idx_v[b], sio[b])
        pltpu.async_copy(wgt_hbm.at[pl.ds(q, _CQ)], wgt_v[b], sio[b])

    def wait_io(b):
        pltpu.make_async_copy(idx_hbm.at[pl.ds(0, _CQ * 4)], idx_v[b], sio[b]).wait()
        pltpu.make_async_copy(wgt_hbm.at[pl.ds(0, _CQ)], wgt_v[b], sio[b]).wait()

    def fire_gathers(b):
        for g in range(_CQ * 4):
            pltpu.async_copy(table_hbm.at[idx_v[b].at[g]],
                             rows_v[b].at[pl.ds(g * 128, 128)], sg[b])

    def wait_gathers(b):
        pltpu.make_async_copy(table_hbm.at[pl.ds(0, _CQ * 512)], rows_v[b], sg[b]).wait()

    def wait_out(b):
        pltpu.make_async_copy(out_v[b], out_hbm.at[pl.ds(q_base, _CQ)], so[b]).wait()

    def compute(b, q0):
        for q in range(_CQ):
            def hbody(h, c2):
                acc0 = jnp.zeros((16,), jnp.float32)
                acc1 = jnp.zeros((16,), jnp.float32)
                for c in range(4):
                    wv = wgt_v[b][q, pl.ds(pl.multiple_of(c * 128 + h * 16, 16), 16)]
                    for k2 in range(16):
                        r = q * 512 + c * 128 + h * 16 + k2
                        w = wv[k2]
                        a0, a1 = plsc.unpack(rows_v[b][r, pl.ds(0, 32)],
                                             format=plsc.PackFormat.INTERLEAVED)
                        acc0 = acc0 + w * a0
                        acc1 = acc1 + w * a1
                o = pl.multiple_of(h * 32, 32)
                out_v[b][q, pl.ds(o, 16)] = acc0
                out_v[b][q, pl.ds(o + 16, 16)] = acc1
                return c2
            lax.fori_loop(0, _NH, hbody, 0)
        pltpu.async_copy(out_v[b], out_hbm.at[pl.ds(q0, _CQ)], so[b])

    issue_io(0, 0)
    wait_io(0)
    fire_gathers(0)
    issue_io(1, 1)

    def iter_body(i, carry):
        for b in (0, 1):
            c = 2 * i + b
            wait_io(1 - b)
            fire_gathers(1 - b)
            wait_gathers(b)
            @pl.when(i >= 1)
            def _():
                wait_out(b)
            compute(b, q_base + c * _CQ)
            issue_io(c + 2, b)
        return carry

    lax.fori_loop(0, _NCHUNK // 2, iter_body, 0)

    wait_io(1)
    wait_gathers(0)
    wait_out(0)
    wait_out(1)


def _sc_sample(table, idx4, wgt):
    mesh = plsc.VectorSubcoreMesh(core_axis_name="c", subcore_axis_name="s")
    return pl.kernel(
        _sc_body,
        out_type=jax.ShapeDtypeStruct((_NQ, _D), jnp.float32),
        mesh=mesh,
        compiler_params=pltpu.CompilerParams(use_tc_tiling_on_sc=False,
                                             needs_layout_passes=False),
        scratch_types=[
            pltpu.VMEM((_CQ * 4, 128), jnp.int32),
            pltpu.VMEM((_CQ * 4, 128), jnp.int32),
            pltpu.VMEM((_CQ, 512), jnp.float32),
            pltpu.VMEM((_CQ, 512), jnp.float32),
            pltpu.VMEM((_CQ * 512, _HD), jnp.bfloat16),
            pltpu.VMEM((_CQ * 512, _HD), jnp.bfloat16),
            pltpu.VMEM((_CQ, _D), jnp.float32),
            pltpu.VMEM((_CQ, _D), jnp.float32),
            pltpu.SemaphoreType.DMA,
            pltpu.SemaphoreType.DMA,
            pltpu.SemaphoreType.DMA,
            pltpu.SemaphoreType.DMA,
            pltpu.SemaphoreType.DMA,
            pltpu.SemaphoreType.DMA,
        ],
    )(table, idx4, wgt)


def kernel(query, reference_points, input_flatten, input_spatial_shapes,
           input_level_start_index, Wv, bv, Wo, bo, Wa, ba, Wout, bout):
    q2 = query.reshape(_NQ, _D)
    rp8 = reference_points.reshape(_NQ, _NL * 2)

    value = _matmul_bias(input_flatten.reshape(_N * _S, _D), Wv, bv,
                         out_dtype=jnp.bfloat16)
    table = value.reshape(_N * _S * _NH, _HD)

    consts = (
        jnp.asarray(_BBASE),
        Wo[:, 0::2], Wo[:, 1::2], Wa,
        bo[0::2][None, :], bo[1::2][None, :], ba[None, :],
        jnp.asarray(_RX8), jnp.asarray(_RY8), jnp.asarray(_BD),
        jnp.asarray(_WI), jnp.asarray(_HI), jnp.asarray(_W8),
        jnp.asarray(_CBASE),
    )
    idx, wgt = _prep(q2, rp8, consts)

    sampled = _sc_sample(table, idx.reshape(_NQ * 4, 128), wgt)
    out = _matmul_bias(sampled, Wout[jnp.asarray(_PERM)], bout)
    return out.reshape(_N, _LQ, _D)


# speedup vs baseline: 135.6804x; 1.0770x over previous
import functools

import numpy as np
import jax
import jax.numpy as jnp
from jax import lax
from jax.experimental import pallas as pl
from jax.experimental.pallas import tpu as pltpu
from jax.experimental.pallas import tpu_sc as plsc

_D = 256
_NH = 8
_NL = 4
_NP = 4
_HD = _D // _NH
_SPAT = np.array([[64, 64], [32, 32], [16, 16], [8, 8]], dtype=np.int64)
_LSTART = np.array([0, 4096, 5120, 5376], dtype=np.int64)
_N = 4
_LQ = 5440
_S = int((_SPAT[:, 0] * _SPAT[:, 1]).sum())
_NQ = _N * _LQ
_NSAMP = _NH * _NL * _NP
_BLK = 128
_GRID = _NQ // _BLK

_NW = 32
_QPW = _NQ // _NW
_CQ = 4
_NCHUNK = _QPW // _CQ

_s = np.arange(_NSAMP)
_lvl = (_s // _NP) % _NL
_head = _s // (_NL * _NP)
_Wl = _SPAT[_lvl, 1].astype(np.int32)
_Hl = _SPAT[_lvl, 0].astype(np.int32)

_WI = _Wl[None, :]
_HI = _Hl[None, :]
_W8 = (_Wl * _NH)[None, :].astype(np.int32)
_CBASE = ((_LSTART[_lvl] * _NH) + _head)[None, :].astype(np.int32)

_RX8 = np.zeros((2 * _NL, _NSAMP), np.float32)
_RY8 = np.zeros((2 * _NL, _NSAMP), np.float32)
for _j in range(_NSAMP):
    _RX8[2 * _lvl[_j], _j] = float(_Wl[_j])
    _RY8[2 * _lvl[_j] + 1, _j] = float(_Hl[_j])

_BD = (( _s[:, None] // (_NL * _NP)) == (_s[None, :] // (_NL * _NP))).astype(np.float32)

_BBASE = ((np.arange(_NQ) // _LQ) * (_S * _NH)).astype(np.int32)[:, None]

_PERM = np.empty(_D, np.int32)
for _k in range(_D):
    _h, _j = divmod(_k, _HD)
    _PERM[_k] = _h * _HD + (2 * _j if _j < 16 else 2 * (_j - 16) + 1)


def _mm_body(x_ref, w_ref, b_ref, o_ref):
    o_ref[...] = (jnp.dot(x_ref[...], w_ref[...],
                          preferred_element_type=jnp.float32)
                  + b_ref[...]).astype(o_ref.dtype)


def _matmul_bias(x, w, b, out_dtype=jnp.float32):
    n, k = x.shape
    m = w.shape[1]
    return pl.pallas_call(
        _mm_body,
        grid=(n // _BLK,),
        in_specs=[
            pl.BlockSpec((_BLK, k), lambda i: (i, 0)),
            pl.BlockSpec((k, m), lambda i: (0, 0)),
            pl.BlockSpec((1, m), lambda i: (0, 0)),
        ],
        out_specs=pl.BlockSpec((_BLK, m), lambda i: (i, 0)),
        out_shape=jax.ShapeDtypeStruct((n, m), out_dtype),
    )(x, w, b[None, :])


def _prep_body(q_ref, rp_ref, bb_ref, wox_ref, woy_ref, wa_ref,
               box_ref, boy_ref, ba_ref, rx_ref, ry_ref, bd_ref,
               wi_ref, hi_ref, w8_ref, cb_ref,
               idx_ref, wgt_ref):
    q = q_ref[...]
    offx = jnp.dot(q, wox_ref[...], preferred_element_type=jnp.float32, precision=jax.lax.Precision.HIGHEST) + box_ref[...]
    offy = jnp.dot(q, woy_ref[...], preferred_element_type=jnp.float32, precision=jax.lax.Precision.HIGHEST) + boy_ref[...]
    rp = rp_ref[...]
    x = jnp.dot(rp, rx_ref[...], preferred_element_type=jnp.float32, precision=jax.lax.Precision.HIGHEST) + offx - 0.5
    y = jnp.dot(rp, ry_ref[...], preferred_element_type=jnp.float32, precision=jax.lax.Precision.HIGHEST) + offy - 0.5

    logits = jnp.dot(q, wa_ref[...], preferred_element_type=jnp.float32) + ba_ref[...]
    m = jnp.max(logits, axis=1, keepdims=True)
    e = jnp.exp(logits - m)
    aw = e / jnp.dot(e, bd_ref[...], preferred_element_type=jnp.float32)

    x0 = jnp.floor(x)
    y0 = jnp.floor(y)
    fx = x - x0
    fy = y - y0
    x0i = x0.astype(jnp.int32)
    y0i = y0.astype(jnp.int32)
    wi = wi_ref[...]
    hi = hi_ref[...]
    w8 = w8_ref[...]
    base = bb_ref[...] + cb_ref[...]
    for ci, (dx, dy) in enumerate(((0, 0), (1, 0), (0, 1), (1, 1))):
        xi = x0i + dx
        yi = y0i + dy
        valid = (xi >= 0) & (xi < wi) & (yi >= 0) & (yi < hi)
        xc = jnp.clip(xi, 0, wi - 1)
        yc = jnp.clip(yi, 0, hi - 1)
        idx_ref[:, ci * 128:(ci + 1) * 128] = base + yc * w8 + xc * _NH
        wx = fx if dx else 1.0 - fx
        wy = fy if dy else 1.0 - fy
        wgt_ref[:, ci * 128:(ci + 1) * 128] = jnp.where(valid, wx * wy * aw, 0.0)


def _prep(q2, rp8, consts):
    (bb, wox, woy, wa, box, boy, ba, rx, ry, bd, wi, hi, w8, cb) = consts
    full = lambda a, b: pl.BlockSpec((a, b), lambda i: (0, 0))
    return pl.pallas_call(
        _prep_body,
        grid=(_GRID,),
        in_specs=[
            pl.BlockSpec((_BLK, _D), lambda i: (i, 0)),
            pl.BlockSpec((_BLK, 8), lambda i: (i, 0)),
            pl.BlockSpec((_BLK, 1), lambda i: (i, 0)),
            full(_D, 128), full(_D, 128), full(_D, 128),
            full(1, 128), full(1, 128), full(1, 128),
            full(8, 128), full(8, 128), full(128, 128),
            full(1, 128), full(1, 128), full(1, 128), full(1, 128),
        ],
        out_specs=[
            pl.BlockSpec((_BLK, 512), lambda i: (i, 0)),
            pl.BlockSpec((_BLK, 512), lambda i: (i, 0)),
        ],
        out_shape=[
            jax.ShapeDtypeStruct((_NQ, 512), jnp.int32),
            jax.ShapeDtypeStruct((_NQ, 512), jnp.float32),
        ],
    )(q2, rp8, bb, wox, woy, wa, box, boy, ba, rx, ry, bd, wi, hi, w8, cb)


def _sc_body(table_hbm, idx_hbm, wgt_hbm, out_hbm,
             idx0, idx1, wgt0, wgt1, rows0, rows1, out0, out1,
             sg0, sg1, sio0, sio1, so0, so1):
    idx_v = (idx0, idx1)
    wgt_v = (wgt0, wgt1)
    rows_v = (rows0, rows1)
    out_v = (out0, out1)
    sg = (sg0, sg1)
    sio = (sio0, sio1)
    so = (so0, so1)
    wid = lax.axis_index("s") * 2 + lax.axis_index("c")
    q_base = wid * _QPW
    qmax = _NQ - _CQ

    def qof(c):
        return jnp.minimum(q_base + c * _CQ, qmax)

    def issue_io(c, b):
        q = qof(c)
        pltpu.async_copy(idx_hbm.at[pl.ds(q * 4, _CQ * 4)], idx_v[b], sio[b])
        pltpu.async_copy(wgt_hbm.at[pl.ds(q, _CQ)], wgt_v[b], sio[b])

    def wait_io(b):
        pltpu.make_async_copy(idx_hbm.at[pl.ds(0, _CQ * 4)], idx_v[b], sio[b]).wait()
        pltpu.make_async_copy(wgt_hbm.at[pl.ds(0, _CQ)], wgt_v[b], sio[b]).wait()

    def fire_gathers(b):
        for g in range(_CQ * 4):
            pltpu.async_copy(table_hbm.at[idx_v[b].at[g]],
                             rows_v[b].at[pl.ds(g * 128, 128)], sg[b])

    def wait_gathers(b):
        pltpu.make_async_copy(table_hbm.at[pl.ds(0, _CQ * 512)], rows_v[b], sg[b]).wait()

    def wait_out(b):
        pltpu.make_async_copy(out_v[b], out_hbm.at[pl.ds(q_base, _CQ)], so[b]).wait()

    def compute(b, q0):
        for q in range(_CQ):
            def hbody(h, c2):
                a0s = [jnp.zeros((16,), jnp.float32) for _ in range(4)]
                a1s = [jnp.zeros((16,), jnp.float32) for _ in range(4)]
                for c in range(4):
                    wv = wgt_v[b][q, pl.ds(pl.multiple_of(c * 128 + h * 16, 16), 16)]
                    for k2 in range(16):
                        r = q * 512 + c * 128 + h * 16 + k2
                        w = wv[k2]
                        a0, a1 = plsc.unpack(rows_v[b][r, pl.ds(0, 32)],
                                             format=plsc.PackFormat.INTERLEAVED)
                        a0s[c] = a0s[c] + w * a0
                        a1s[c] = a1s[c] + w * a1
                o = pl.multiple_of(h * 32, 32)
                out_v[b][q, pl.ds(o, 16)] = (a0s[0] + a0s[1]) + (a0s[2] + a0s[3])
                out_v[b][q, pl.ds(o + 16, 16)] = (a1s[0] + a1s[1]) + (a1s[2] + a1s[3])
                return c2
            lax.fori_loop(0, _NH, hbody, 0)
        pltpu.async_copy(out_v[b], out_hbm.at[pl.ds(q0, _CQ)], so[b])

    issue_io(0, 0)
    wait_io(0)
    fire_gathers(0)
    issue_io(1, 1)

    def iter_body(i, carry):
        for b in (0, 1):
            c = 2 * i + b
            wait_io(1 - b)
            fire_gathers(1 - b)
            wait_gathers(b)
            @pl.when(i >= 1)
            def _():
                wait_out(b)
            compute(b, q_base + c * _CQ)
            issue_io(c + 2, b)
        return carry

    lax.fori_loop(0, _NCHUNK // 2, iter_body, 0)

    wait_io(1)
    wait_gathers(0)
    wait_out(0)
    wait_out(1)


def _sc_sample(table, idx4, wgt):
    mesh = plsc.VectorSubcoreMesh(core_axis_name="c", subcore_axis_name="s")
    return pl.kernel(
        _sc_body,
        out_type=jax.ShapeDtypeStruct((_NQ, _D), jnp.float32),
        mesh=mesh,
        compiler_params=pltpu.CompilerParams(use_tc_tiling_on_sc=False,
                                             needs_layout_passes=False),
        scratch_types=[
            pltpu.VMEM((_CQ * 4, 128), jnp.int32),
            pltpu.VMEM((_CQ * 4, 128), jnp.int32),
            pltpu.VMEM((_CQ, 512), jnp.float32),
            pltpu.VMEM((_CQ, 512), jnp.float32),
            pltpu.VMEM((_CQ * 512, _HD), jnp.bfloat16),
            pltpu.VMEM((_CQ * 512, _HD), jnp.bfloat16),
            pltpu.VMEM((_CQ, _D), jnp.float32),
            pltpu.VMEM((_CQ, _D), jnp.float32),
            pltpu.SemaphoreType.DMA,
            pltpu.SemaphoreType.DMA,
            pltpu.SemaphoreType.DMA,
            pltpu.SemaphoreType.DMA,
            pltpu.SemaphoreType.DMA,
            pltpu.SemaphoreType.DMA,
        ],
    )(table, idx4, wgt)


def kernel(query, reference_points, input_flatten, input_spatial_shapes,
           input_level_start_index, Wv, bv, Wo, bo, Wa, ba, Wout, bout):
    q2 = query.reshape(_NQ, _D)
    rp8 = reference_points.reshape(_NQ, _NL * 2)

    value = _matmul_bias(input_flatten.reshape(_N * _S, _D), Wv, bv,
                         out_dtype=jnp.bfloat16)
    table = value.reshape(_N * _S * _NH, _HD)

    consts = (
        jnp.asarray(_BBASE),
        Wo[:, 0::2], Wo[:, 1::2], Wa,
        bo[0::2][None, :], bo[1::2][None, :], ba[None, :],
        jnp.asarray(_RX8), jnp.asarray(_RY8), jnp.asarray(_BD),
        jnp.asarray(_WI), jnp.asarray(_HI), jnp.asarray(_W8),
        jnp.asarray(_CBASE),
    )
    idx, wgt = _prep(q2, rp8, consts)

    sampled = _sc_sample(table, idx.reshape(_NQ * 4, 128), wgt)
    out = _matmul_bias(sampled, Wout[jnp.asarray(_PERM)], bout)
    return out.reshape(_N, _LQ, _D)


# speedup vs baseline: 144.2332x; 1.0630x over previous
import functools

import numpy as np
import jax
import jax.numpy as jnp
from jax import lax
from jax.experimental import pallas as pl
from jax.experimental.pallas import tpu as pltpu
from jax.experimental.pallas import tpu_sc as plsc

_D = 256
_NH = 8
_NL = 4
_NP = 4
_HD = _D // _NH
_SPAT = np.array([[64, 64], [32, 32], [16, 16], [8, 8]], dtype=np.int64)
_LSTART = np.array([0, 4096, 5120, 5376], dtype=np.int64)
_N = 4
_LQ = 5440
_S = int((_SPAT[:, 0] * _SPAT[:, 1]).sum())
_NQ = _N * _LQ
_NSAMP = _NH * _NL * _NP
_BLK = 128
_GRID = _NQ // _BLK

_NW = 32
_QPW = _NQ // _NW
_CQ = 4
_NCHUNK = _QPW // _CQ

_s = np.arange(_NSAMP)
_lvl = (_s // _NP) % _NL
_head = _s // (_NL * _NP)
_Wl = _SPAT[_lvl, 1].astype(np.int32)
_Hl = _SPAT[_lvl, 0].astype(np.int32)

_WI = _Wl[None, :]
_HI = _Hl[None, :]
_W8 = (_Wl * _NH)[None, :].astype(np.int32)
_CBASE = ((_LSTART[_lvl] * _NH) + _head)[None, :].astype(np.int32)

_RX8 = np.zeros((2 * _NL, _NSAMP), np.float32)
_RY8 = np.zeros((2 * _NL, _NSAMP), np.float32)
for _j in range(_NSAMP):
    _RX8[2 * _lvl[_j], _j] = float(_Wl[_j])
    _RY8[2 * _lvl[_j] + 1, _j] = float(_Hl[_j])

_BD = (( _s[:, None] // (_NL * _NP)) == (_s[None, :] // (_NL * _NP))).astype(np.float32)

_BBASE = ((np.arange(_NQ) // _LQ) * (_S * _NH)).astype(np.int32)[:, None]

_PERM = np.empty(_D, np.int32)
for _k in range(_D):
    _h, _j = divmod(_k, _HD)
    _PERM[_k] = _h * _HD + (2 * _j if _j < 16 else 2 * (_j - 16) + 1)


def _mm_body(x_ref, w_ref, b_ref, o_ref):
    o_ref[...] = (jnp.dot(x_ref[...], w_ref[...],
                          preferred_element_type=jnp.float32)
                  + b_ref[...]).astype(o_ref.dtype)


def _matmul_bias(x, w, b, out_dtype=jnp.float32):
    n, k = x.shape
    m = w.shape[1]
    return pl.pallas_call(
        _mm_body,
        grid=(n // _BLK,),
        in_specs=[
            pl.BlockSpec((_BLK, k), lambda i: (i, 0)),
            pl.BlockSpec((k, m), lambda i: (0, 0)),
            pl.BlockSpec((1, m), lambda i: (0, 0)),
        ],
        out_specs=pl.BlockSpec((_BLK, m), lambda i: (i, 0)),
        out_shape=jax.ShapeDtypeStruct((n, m), out_dtype),
    )(x, w, b[None, :])


def _prep_body(q_ref, rp_ref, bb_ref, flat_ref, wv_ref, bv_ref,
               wox_ref, woy_ref, wa_ref,
               box_ref, boy_ref, ba_ref, rx_ref, ry_ref, bd_ref,
               wi_ref, hi_ref, w8_ref, cb_ref,
               val_ref, idx_ref, wgt_ref):
    val_ref[...] = (jnp.dot(flat_ref[...], wv_ref[...],
                            preferred_element_type=jnp.float32)
                    + bv_ref[...]).astype(val_ref.dtype)
    q = q_ref[...]
    offx = jnp.dot(q, wox_ref[...], preferred_element_type=jnp.float32, precision=jax.lax.Precision.HIGHEST) + box_ref[...]
    offy = jnp.dot(q, woy_ref[...], preferred_element_type=jnp.float32, precision=jax.lax.Precision.HIGHEST) + boy_ref[...]
    rp = rp_ref[...]
    x = jnp.dot(rp, rx_ref[...], preferred_element_type=jnp.float32, precision=jax.lax.Precision.HIGHEST) + offx - 0.5
    y = jnp.dot(rp, ry_ref[...], preferred_element_type=jnp.float32, precision=jax.lax.Precision.HIGHEST) + offy - 0.5

    logits = jnp.dot(q, wa_ref[...], preferred_element_type=jnp.float32) + ba_ref[...]
    m = jnp.max(logits, axis=1, keepdims=True)
    e = jnp.exp(logits - m)
    aw = e / jnp.dot(e, bd_ref[...], preferred_element_type=jnp.float32)

    x0 = jnp.floor(x)
    y0 = jnp.floor(y)
    fx = x - x0
    fy = y - y0
    x0i = x0.astype(jnp.int32)
    y0i = y0.astype(jnp.int32)
    wi = wi_ref[...]
    hi = hi_ref[...]
    w8 = w8_ref[...]
    base = bb_ref[...] + cb_ref[...]
    for ci, (dx, dy) in enumerate(((0, 0), (1, 0), (0, 1), (1, 1))):
        xi = x0i + dx
        yi = y0i + dy
        valid = (xi >= 0) & (xi < wi) & (yi >= 0) & (yi < hi)
        xc = jnp.clip(xi, 0, wi - 1)
        yc = jnp.clip(yi, 0, hi - 1)
        idx_ref[:, ci * 128:(ci + 1) * 128] = base + yc * w8 + xc * _NH
        wx = fx if dx else 1.0 - fx
        wy = fy if dy else 1.0 - fy
        wgt_ref[:, ci * 128:(ci + 1) * 128] = jnp.where(valid, wx * wy * aw, 0.0)


def _prep(q2, rp8, flat, Wv, bv, consts):
    (bb, wox, woy, wa, box, boy, ba, rx, ry, bd, wi, hi, w8, cb) = consts
    full = lambda a, b: pl.BlockSpec((a, b), lambda i: (0, 0))
    return pl.pallas_call(
        _prep_body,
        grid=(_GRID,),
        in_specs=[
            pl.BlockSpec((_BLK, _D), lambda i: (i, 0)),
            pl.BlockSpec((_BLK, 8), lambda i: (i, 0)),
            pl.BlockSpec((_BLK, 1), lambda i: (i, 0)),
            pl.BlockSpec((_BLK, _D), lambda i: (i, 0)),
            full(_D, _D), full(1, _D),
            full(_D, 128), full(_D, 128), full(_D, 128),
            full(1, 128), full(1, 128), full(1, 128),
            full(8, 128), full(8, 128), full(128, 128),
            full(1, 128), full(1, 128), full(1, 128), full(1, 128),
        ],
        out_specs=[
            pl.BlockSpec((_BLK, _D), lambda i: (i, 0)),
            pl.BlockSpec((_BLK, 512), lambda i: (i, 0)),
            pl.BlockSpec((_BLK, 512), lambda i: (i, 0)),
        ],
        out_shape=[
            jax.ShapeDtypeStruct((_NQ, _D), jnp.bfloat16),
            jax.ShapeDtypeStruct((_NQ, 512), jnp.int32),
            jax.ShapeDtypeStruct((_NQ, 512), jnp.float32),
        ],
    )(q2, rp8, bb, flat, Wv, bv[None, :], wox, woy, wa, box, boy, ba,
      rx, ry, bd, wi, hi, w8, cb)


def _sc_body(table_hbm, idx_hbm, wgt_hbm, out_hbm,
             idx0, idx1, wgt0, wgt1, rows0, rows1, out0, out1,
             sg0, sg1, sio0, sio1, so0, so1):
    idx_v = (idx0, idx1)
    wgt_v = (wgt0, wgt1)
    rows_v = (rows0, rows1)
    out_v = (out0, out1)
    sg = (sg0, sg1)
    sio = (sio0, sio1)
    so = (so0, so1)
    wid = lax.axis_index("s") * 2 + lax.axis_index("c")
    q_base = wid * _QPW
    qmax = _NQ - _CQ

    def qof(c):
        return jnp.minimum(q_base + c * _CQ, qmax)

    def issue_io(c, b):
        q = qof(c)
        pltpu.async_copy(idx_hbm.at[pl.ds(q * 4, _CQ * 4)], idx_v[b], sio[b])
        pltpu.async_copy(wgt_hbm.at[pl.ds(q, _CQ)], wgt_v[b], sio[b])

    def wait_io(b):
        pltpu.make_async_copy(idx_hbm.at[pl.ds(0, _CQ * 4)], idx_v[b], sio[b]).wait()
        pltpu.make_async_copy(wgt_hbm.at[pl.ds(0, _CQ)], wgt_v[b], sio[b]).wait()

    def fire_gathers(b):
        for g in range(_CQ * 4):
            pltpu.async_copy(table_hbm.at[idx_v[b].at[g]],
                             rows_v[b].at[pl.ds(g * 128, 128)], sg[b])

    def wait_gathers(b):
        pltpu.make_async_copy(table_hbm.at[pl.ds(0, _CQ * 512)], rows_v[b], sg[b]).wait()

    def wait_out(b):
        pltpu.make_async_copy(out_v[b], out_hbm.at[pl.ds(q_base, _CQ)], so[b]).wait()

    def compute(b, q0):
        for q in range(_CQ):
            def hbody(h, c2):
                a0s = [jnp.zeros((16,), jnp.float32) for _ in range(4)]
                a1s = [jnp.zeros((16,), jnp.float32) for _ in range(4)]
                for c in range(4):
                    wv = wgt_v[b][q, pl.ds(pl.multiple_of(c * 128 + h * 16, 16), 16)]
                    for k2 in range(16):
                        r = q * 512 + c * 128 + h * 16 + k2
                        w = wv[k2]
                        a0, a1 = plsc.unpack(rows_v[b][r, pl.ds(0, 32)],
                                             format=plsc.PackFormat.INTERLEAVED)
                        a0s[c] = a0s[c] + w * a0
                        a1s[c] = a1s[c] + w * a1
                o = pl.multiple_of(h * 32, 32)
                out_v[b][q, pl.ds(o, 16)] = (a0s[0] + a0s[1]) + (a0s[2] + a0s[3])
                out_v[b][q, pl.ds(o + 16, 16)] = (a1s[0] + a1s[1]) + (a1s[2] + a1s[3])
                return c2
            lax.fori_loop(0, _NH, hbody, 0)
        pltpu.async_copy(out_v[b], out_hbm.at[pl.ds(q0, _CQ)], so[b])

    issue_io(0, 0)
    wait_io(0)
    fire_gathers(0)
    issue_io(1, 1)

    def iter_body(i, carry):
        for b in (0, 1):
            c = 2 * i + b
            wait_io(1 - b)
            fire_gathers(1 - b)
            wait_gathers(b)
            @pl.when(i >= 1)
            def _():
                wait_out(b)
            compute(b, q_base + c * _CQ)
            issue_io(c + 2, b)
        return carry

    lax.fori_loop(0, _NCHUNK // 2, iter_body, 0)

    wait_io(1)
    wait_gathers(0)
    wait_out(0)
    wait_out(1)


def _sc_sample(table, idx4, wgt):
    mesh = plsc.VectorSubcoreMesh(core_axis_name="c", subcore_axis_name="s")
    return pl.kernel(
        _sc_body,
        out_type=jax.ShapeDtypeStruct((_NQ, _D), jnp.float32),
        mesh=mesh,
        compiler_params=pltpu.CompilerParams(use_tc_tiling_on_sc=False,
                                             needs_layout_passes=False),
        scratch_types=[
            pltpu.VMEM((_CQ * 4, 128), jnp.int32),
            pltpu.VMEM((_CQ * 4, 128), jnp.int32),
            pltpu.VMEM((_CQ, 512), jnp.float32),
            pltpu.VMEM((_CQ, 512), jnp.float32),
            pltpu.VMEM((_CQ * 512, _HD), jnp.bfloat16),
            pltpu.VMEM((_CQ * 512, _HD), jnp.bfloat16),
            pltpu.VMEM((_CQ, _D), jnp.float32),
            pltpu.VMEM((_CQ, _D), jnp.float32),
            pltpu.SemaphoreType.DMA,
            pltpu.SemaphoreType.DMA,
            pltpu.SemaphoreType.DMA,
            pltpu.SemaphoreType.DMA,
            pltpu.SemaphoreType.DMA,
            pltpu.SemaphoreType.DMA,
        ],
    )(table, idx4, wgt)


def kernel(query, reference_points, input_flatten, input_spatial_shapes,
           input_level_start_index, Wv, bv, Wo, bo, Wa, ba, Wout, bout):
    q2 = query.reshape(_NQ, _D)
    rp8 = reference_points.reshape(_NQ, _NL * 2)

    consts = (
        jnp.asarray(_BBASE),
        Wo[:, 0::2], Wo[:, 1::2], Wa,
        bo[0::2][None, :], bo[1::2][None, :], ba[None, :],
        jnp.asarray(_RX8), jnp.asarray(_RY8), jnp.asarray(_BD),
        jnp.asarray(_WI), jnp.asarray(_HI), jnp.asarray(_W8),
        jnp.asarray(_CBASE),
    )
    value, idx, wgt = _prep(q2, rp8, input_flatten.reshape(_N * _S, _D),
                            Wv, bv, consts)
    table = value.reshape(_N * _S * _NH, _HD)

    sampled = _sc_sample(table, idx.reshape(_NQ * 4, 128), wgt)
    out = _matmul_bias(sampled, Wout[jnp.asarray(_PERM)], bout)
    return out.reshape(_N, _LQ, _D)
